# baseline (speedup 1.0000x reference)
import jax
import jax.numpy as jnp
from jax.experimental import pallas as pl
from jax.experimental.pallas import tpu as pltpu

_T, _E, _D_MODEL, _D_FF = 128, 16, 768, 1536
_NB = 3


def _start(hbm, buf, sems, m, e, s):
    pltpu.make_async_copy(hbm.at[e], buf.at[s], sems.at[m, s]).start()


def _wait(hbm, buf, sems, m, e, s):
    pltpu.make_async_copy(hbm.at[e], buf.at[s], sems.at[m, s]).wait()


def _moe_kernel(ids_ref, x_ref, w1_hbm, w3_hbm, w2_hbm, out_ref,
                w1_buf, w3_buf, w2_buf, sems):
    x = x_ref[...]
    ids = ids_ref[...]

    def issue(e, s):
        _start(w1_hbm, w1_buf, sems, 0, e, s)
        _start(w3_hbm, w3_buf, sems, 1, e, s)
        _start(w2_hbm, w2_buf, sems, 2, e, s)

    issue(0, 0)
    issue(1, 1)

    out_ref[...] = jnp.zeros_like(out_ref)

    for e in range(_E):
        s = e % _NB
        if e + 2 < _E:
            issue(e + 2, (e + 2) % _NB)

        xm = jnp.where(ids == e, x, 0.0)

        _wait(w1_hbm, w1_buf, sems, 0, e, s)
        g = jax.lax.dot_general(xm, w1_buf[s], (((1,), (1,)), ((), ())),
                                preferred_element_type=jnp.float32)
        _wait(w3_hbm, w3_buf, sems, 1, e, s)
        u = jax.lax.dot_general(xm, w3_buf[s], (((1,), (1,)), ((), ())),
                                preferred_element_type=jnp.float32)
        h = (g * jax.nn.sigmoid(g)) * u
        _wait(w2_hbm, w2_buf, sems, 2, e, s)
        out_ref[...] += jax.lax.dot_general(h, w2_buf[s],
                                            (((1,), (1,)), ((), ())),
                                            preferred_element_type=jnp.float32)


def kernel(x, token_expert_ids, w1, w3, w2):
    ids = token_expert_ids.astype(jnp.int32).reshape(_T, 1)
    return pl.pallas_call(
        _moe_kernel,
        in_specs=[
            pl.BlockSpec(memory_space=pltpu.VMEM),
            pl.BlockSpec(memory_space=pltpu.VMEM),
            pl.BlockSpec(memory_space=pl.ANY),
            pl.BlockSpec(memory_space=pl.ANY),
            pl.BlockSpec(memory_space=pl.ANY),
        ],
        out_specs=pl.BlockSpec(memory_space=pltpu.VMEM),
        out_shape=jax.ShapeDtypeStruct((_T, _D_MODEL), jnp.float32),
        scratch_shapes=[
            pltpu.VMEM((_NB, _D_FF, _D_MODEL), jnp.float32),
            pltpu.VMEM((_NB, _D_FF, _D_MODEL), jnp.float32),
            pltpu.VMEM((_NB, _D_MODEL, _D_FF), jnp.float32),
            pltpu.SemaphoreType.DMA((3, _NB)),
        ],
        compiler_params=pltpu.CompilerParams(
            vmem_limit_bytes=100 * 1024 * 1024,
        ),
    )(ids, x, w1, w3, w2)


# speedup vs baseline: 1.0036x; 1.0036x over previous
import jax
import jax.numpy as jnp
from jax.experimental import pallas as pl
from jax.experimental.pallas import tpu as pltpu

_T, _E, _D_MODEL, _D_FF = 128, 16, 768, 1536
_NB = 2


def _start(hbm, buf, sems, m, e, s):
    pltpu.make_async_copy(hbm.at[e], buf.at[s], sems.at[m, s]).start()


def _wait(hbm, buf, sems, m, e, s):
    pltpu.make_async_copy(hbm.at[e], buf.at[s], sems.at[m, s]).wait()


def _moe_kernel(ids_ref, x_ref, w1_hbm, w3_hbm, w2_hbm, out_ref,
                w1_buf, w3_buf, w2_buf, sems):
    x = x_ref[...]
    ids = ids_ref[...]

    def issue(e, s):
        _start(w1_hbm, w1_buf, sems, 0, e, s)
        _start(w3_hbm, w3_buf, sems, 1, e, s)
        _start(w2_hbm, w2_buf, sems, 2, e, s)

    issue(0, 0)

    out_ref[...] = jnp.zeros_like(out_ref)

    for e in range(_E):
        s = e % _NB
        if e + 1 < _E:
            issue(e + 1, (e + 1) % _NB)

        xm = jnp.where(ids == e, x, 0.0)

        _wait(w1_hbm, w1_buf, sems, 0, e, s)
        g = jax.lax.dot_general(xm, w1_buf[s], (((1,), (1,)), ((), ())),
                                preferred_element_type=jnp.float32)
        _wait(w3_hbm, w3_buf, sems, 1, e, s)
        u = jax.lax.dot_general(xm, w3_buf[s], (((1,), (1,)), ((), ())),
                                preferred_element_type=jnp.float32)
        h = (g * jax.nn.sigmoid(g)) * u
        _wait(w2_hbm, w2_buf, sems, 2, e, s)
        out_ref[...] += jax.lax.dot_general(h, w2_buf[s],
                                            (((1,), (1,)), ((), ())),
                                            preferred_element_type=jnp.float32)


def kernel(x, token_expert_ids, w1, w3, w2):
    ids = token_expert_ids.astype(jnp.int32).reshape(_T, 1)
    return pl.pallas_call(
        _moe_kernel,
        in_specs=[
            pl.BlockSpec(memory_space=pltpu.VMEM),
            pl.BlockSpec(memory_space=pltpu.VMEM),
            pl.BlockSpec(memory_space=pl.ANY),
            pl.BlockSpec(memory_space=pl.ANY),
            pl.BlockSpec(memory_space=pl.ANY),
        ],
        out_specs=pl.BlockSpec(memory_space=pltpu.VMEM),
        out_shape=jax.ShapeDtypeStruct((_T, _D_MODEL), jnp.float32),
        scratch_shapes=[
            pltpu.VMEM((_NB, _D_FF, _D_MODEL), jnp.float32),
            pltpu.VMEM((_NB, _D_FF, _D_MODEL), jnp.float32),
            pltpu.VMEM((_NB, _D_MODEL, _D_FF), jnp.float32),
            pltpu.SemaphoreType.DMA((3, _NB)),
        ],
        compiler_params=pltpu.CompilerParams(
            vmem_limit_bytes=100 * 1024 * 1024,
        ),
    )(ids, x, w1, w3, w2)
